# baseline (speedup 1.0000x reference)
import functools

import jax
import jax.numpy as jnp
from jax import lax
from jax.experimental import pallas as pl
from jax.experimental.pallas import tpu as pltpu
from jax.experimental.pallas import tpu_sc as plsc

_B = 4096
_V = 100000
_D = 16
_VG = _V // 8 + 4
_NC = 2
_NS = 16
_NW = _NC * _NS
_CHUNK = _B // _NW


def _tc_repack(in_ref, out_ref):
    x = in_ref[0]
    z = jnp.reshape(x, (16, 512, 8))
    z = jnp.transpose(z, (1, 2, 0))
    out_ref[0] = jnp.reshape(z, (512, 128))


def _repack(tbl_t, n_tables):
    return pl.pallas_call(
        _tc_repack,
        grid=(n_tables, 25),
        in_specs=[pl.BlockSpec((1, 16, 4096), lambda t, c: (t, 0, c))],
        out_specs=pl.BlockSpec((1, 512, 128), lambda t, c: (t, c, 0)),
        out_shape=jax.ShapeDtypeStruct((n_tables, _VG, 128), jnp.float32),
    )(tbl_t)


def _sc_gather(ut, it, swt, uid, uc1, uc2, uc3, iid, ic1, swi,
               u_out, i_out, sw_out,
               idx_raw, g_idx, rows, sw_rows, sem):
    wid = lax.axis_index("s") * _NC + lax.axis_index("c")
    base = wid * _CHUNK

    u_srcs = (uid, uc1, uc2, uc3)
    for t in range(4):
        pltpu.sync_copy(u_srcs[t].at[pl.ds(base, _CHUNK)], idx_raw)
        for j in range(_CHUNK // 16):
            sl = pl.ds(16 * j, 16)
            g_idx[sl] = idx_raw[sl] >> 3
        pltpu.async_copy(ut.at[t].at[g_idx], rows, sem).wait()
        pltpu.sync_copy(rows, u_out.at[t, pl.ds(base, _CHUNK)])

    i_srcs = (iid, ic1)
    for t in range(2):
        pltpu.sync_copy(i_srcs[t].at[pl.ds(base, _CHUNK)], idx_raw)
        for j in range(_CHUNK // 16):
            sl = pl.ds(16 * j, 16)
            g_idx[sl] = idx_raw[sl] >> 3
        pltpu.async_copy(it.at[t].at[g_idx], rows, sem).wait()
        pltpu.sync_copy(rows, i_out.at[t, pl.ds(base, _CHUNK)])

    pltpu.sync_copy(swi.at[pl.ds(base, _CHUNK)], idx_raw)
    pltpu.async_copy(swt.at[idx_raw], sw_rows, sem).wait()
    pltpu.sync_copy(sw_rows, sw_out.at[pl.ds(base, _CHUNK)])


@functools.cache
def _gather_call():
    return pl.kernel(
        _sc_gather,
        mesh=plsc.VectorSubcoreMesh(core_axis_name="c", subcore_axis_name="s"),
        compiler_params=pltpu.CompilerParams(use_tc_tiling_on_sc=False),
        out_type=[
            jax.ShapeDtypeStruct((4, _B, 128), jnp.float32),
            jax.ShapeDtypeStruct((2, _B, 128), jnp.float32),
            jax.ShapeDtypeStruct((_B, 1), jnp.float32),
        ],
        scratch_types=[
            pltpu.VMEM((_CHUNK,), jnp.int32),
            pltpu.VMEM((_CHUNK,), jnp.int32),
            pltpu.VMEM((_CHUNK, 128), jnp.float32),
            pltpu.VMEM((_CHUNK, 1), jnp.float32),
            pltpu.SemaphoreType.DMA,
        ],
    )


def _bn_relu(h, g, be):
    mu = jnp.mean(h, axis=0, keepdims=True)
    var = jnp.mean((h - mu) ** 2, axis=0, keepdims=True)
    return jnp.maximum((h - mu) * lax.rsqrt(var + 1e-5) * g + be, 0.0)


def _select(grp, rem_col, slot):
    m = jnp.where(slot == rem_col, grp, 0.0)
    acc = m[:, 0:16]
    for k in range(1, 8):
        acc = acc + m[:, 16 * k:16 * (k + 1)]
    return acc


_SEL_BLK = 512


def _tc_select(u4, i2, rem, ue_out, ie_out):
    slot = lax.broadcasted_iota(jnp.int32, (_SEL_BLK, 128), 1) // 16
    ue_out[...] = jnp.concatenate(
        [_select(u4[t], rem[:, t:t + 1], slot) for t in range(4)], axis=1)
    ie_out[...] = jnp.concatenate(
        [_select(i2[t], rem[:, 4 + t:5 + t], slot) for t in range(2)], axis=1)


_select_call = pl.pallas_call(
    _tc_select,
    grid=(_B // _SEL_BLK,),
    in_specs=[
        pl.BlockSpec((4, _SEL_BLK, 128), lambda b: (0, b, 0)),
        pl.BlockSpec((2, _SEL_BLK, 128), lambda b: (0, b, 0)),
        pl.BlockSpec((_SEL_BLK, 6), lambda b: (b, 0)),
    ],
    out_specs=[
        pl.BlockSpec((_SEL_BLK, 64), lambda b: (b, 0)),
        pl.BlockSpec((_SEL_BLK, 32), lambda b: (b, 0)),
    ],
    out_shape=[
        jax.ShapeDtypeStruct((_B, 64), jnp.float32),
        jax.ShapeDtypeStruct((_B, 32), jnp.float32),
    ],
)


def _tc_dense(ue_ref, ie_ref, sw,
              uW1, ub1, ug1, ube1, uW2, ub2, ug2, ube2,
              iW1, ib1, ig1, ibe1, iW2, ib2, ig2, ibe2,
              out):
    ue = ue_ref[...]
    ie = ie_ref[...]

    hu = jnp.dot(ue, uW1[...], preferred_element_type=jnp.float32) + ub1[...]
    hu = _bn_relu(hu, ug1[...], ube1[...])
    hu = jnp.dot(hu, uW2[...], preferred_element_type=jnp.float32) + ub2[...]
    hu = _bn_relu(hu, ug2[...], ube2[...])

    hi = jnp.dot(ie, iW1[...], preferred_element_type=jnp.float32) + ib1[...]
    hi = _bn_relu(hi, ig1[...], ibe1[...])
    hi = jnp.dot(hi, iW2[...], preferred_element_type=jnp.float32) + ib2[...]
    hi = _bn_relu(hi, ig2[...], ibe2[...])

    un = hu / jnp.maximum(
        jnp.sqrt(jnp.sum(hu * hu, axis=1, keepdims=True)), 1e-8)
    im = hi / jnp.maximum(
        jnp.sqrt(jnp.sum(hi * hi, axis=1, keepdims=True)), 1e-8)

    lsw = jnp.log(sw[...])

    cols = []
    for k in range(4):
        if k:
            imr = jnp.concatenate([im[k:], im[:k]], axis=0)
            swr = jnp.concatenate([lsw[k:], lsw[:k]], axis=0)
        else:
            imr, swr = im, lsw
        cols.append(jnp.sum(un * imr, axis=1, keepdims=True) - swr)
    out[...] = jnp.concatenate(cols, axis=1)


_dense_call = pl.pallas_call(
    _tc_dense,
    out_shape=jax.ShapeDtypeStruct((_B, 4), jnp.float32),
)


def kernel(user_id, user_cat1, user_cat2, user_cat3, item_id, item_cat1,
           sw_idx, user_tables, item_tables, sw_table,
           u_W1, u_b1, u_g1, u_be1, u_W2, u_b2, u_g2, u_be2,
           i_W1, i_b1, i_g1, i_be1, i_W2, i_b2, i_g2, i_be2):
    ut_g = _repack(jnp.transpose(user_tables, (0, 2, 1)), 4)
    it_g = _repack(jnp.transpose(item_tables, (0, 2, 1)), 2)
    u4, i2, sw = _gather_call()(ut_g, it_g, sw_table,
                                user_id, user_cat1, user_cat2, user_cat3,
                                item_id, item_cat1, sw_idx)
    rem = jnp.stack([user_id, user_cat1, user_cat2, user_cat3,
                     item_id, item_cat1], axis=1) & 7
    ue, ie = _select_call(u4, i2, rem)
    out = _dense_call(
        ue, ie, sw,
        u_W1, u_b1.reshape(1, -1), u_g1.reshape(1, -1), u_be1.reshape(1, -1),
        u_W2, u_b2.reshape(1, -1), u_g2.reshape(1, -1), u_be2.reshape(1, -1),
        i_W1, i_b1.reshape(1, -1), i_g1.reshape(1, -1), i_be1.reshape(1, -1),
        i_W2, i_b2.reshape(1, -1), i_g2.reshape(1, -1), i_be2.reshape(1, -1))
    return out


# speedup vs baseline: 4.4916x; 4.4916x over previous
import functools

import jax
import jax.numpy as jnp
from jax import lax
from jax.experimental import pallas as pl
from jax.experimental.pallas import tpu as pltpu
from jax.experimental.pallas import tpu_sc as plsc

_B = 4096
_V = 100000
_D = 16
_NC = 2
_NS = 16
_NW = _NC * _NS
_CHUNK = _B // _NW


def _emb_gather(tbl, idx_buf, rows_flat, sem):
    copies = [
        pltpu.async_copy(
            tbl.at[idx_buf.at[pl.ds(128 * k, 128)]],
            rows_flat.at[pl.ds(128 * k, 128)], sem)
        for k in range(_CHUNK // 8)
    ]
    for c in copies:
        c.wait()


def _sc_gather(ut, it, swt, uwidx, iwidx, swi,
               u_out, i_out, sw_out,
               idx_raw, idx_buf, rows_flat, sw_rows, sem):
    wid = lax.axis_index("s") * _NC + lax.axis_index("c")
    base = wid * _CHUNK

    for t in range(4):
        pltpu.sync_copy(
            uwidx.at[t, pl.ds(base * _D, _CHUNK * _D)], idx_buf)
        _emb_gather(ut, idx_buf, rows_flat, sem)
        pltpu.sync_copy(rows_flat,
                        u_out.at[t, pl.ds(base * _D, _CHUNK * _D)])

    for t in range(2):
        pltpu.sync_copy(
            iwidx.at[t, pl.ds(base * _D, _CHUNK * _D)], idx_buf)
        _emb_gather(it, idx_buf, rows_flat, sem)
        pltpu.sync_copy(rows_flat,
                        i_out.at[t, pl.ds(base * _D, _CHUNK * _D)])

    pltpu.sync_copy(swi.at[pl.ds(base, _CHUNK)], idx_raw)
    pltpu.async_copy(swt.at[idx_raw], sw_rows, sem).wait()
    pltpu.sync_copy(sw_rows, sw_out.at[pl.ds(base, _CHUNK)])


@functools.cache
def _gather_call():
    return pl.kernel(
        _sc_gather,
        mesh=plsc.VectorSubcoreMesh(core_axis_name="c", subcore_axis_name="s"),
        compiler_params=pltpu.CompilerParams(use_tc_tiling_on_sc=False),
        out_type=[
            jax.ShapeDtypeStruct((4, _B * _D), jnp.float32),
            jax.ShapeDtypeStruct((2, _B * _D), jnp.float32),
            jax.ShapeDtypeStruct((_B,), jnp.float32),
        ],
        scratch_types=[
            pltpu.VMEM((_CHUNK,), jnp.int32),
            pltpu.VMEM((_CHUNK * _D,), jnp.int32),
            pltpu.VMEM((_CHUNK * _D,), jnp.float32),
            pltpu.VMEM((_CHUNK,), jnp.float32),
            pltpu.SemaphoreType.DMA,
        ],
    )


def _bn_relu(h, g, be):
    mu = jnp.mean(h, axis=0, keepdims=True)
    var = jnp.mean((h - mu) ** 2, axis=0, keepdims=True)
    return jnp.maximum((h - mu) * lax.rsqrt(var + 1e-5) * g + be, 0.0)


def _tc_dense(u4, i2, sw,
              uW1, ub1, ug1, ube1, uW2, ub2, ug2, ube2,
              iW1, ib1, ig1, ibe1, iW2, ib2, ig2, ibe2,
              out):
    ue = jnp.concatenate([u4[t] for t in range(4)], axis=1)
    ie = jnp.concatenate([i2[t] for t in range(2)], axis=1)

    hu = jnp.dot(ue, uW1[...], preferred_element_type=jnp.float32) + ub1[...]
    hu = _bn_relu(hu, ug1[...], ube1[...])
    hu = jnp.dot(hu, uW2[...], preferred_element_type=jnp.float32) + ub2[...]
    hu = _bn_relu(hu, ug2[...], ube2[...])

    hi = jnp.dot(ie, iW1[...], preferred_element_type=jnp.float32) + ib1[...]
    hi = _bn_relu(hi, ig1[...], ibe1[...])
    hi = jnp.dot(hi, iW2[...], preferred_element_type=jnp.float32) + ib2[...]
    hi = _bn_relu(hi, ig2[...], ibe2[...])

    un = hu / jnp.maximum(
        jnp.sqrt(jnp.sum(hu * hu, axis=1, keepdims=True)), 1e-8)
    im = hi / jnp.maximum(
        jnp.sqrt(jnp.sum(hi * hi, axis=1, keepdims=True)), 1e-8)

    lsw = jnp.log(sw[...])

    cols = []
    for k in range(4):
        if k:
            imr = jnp.concatenate([im[k:], im[:k]], axis=0)
            swr = jnp.concatenate([lsw[k:], lsw[:k]], axis=0)
        else:
            imr, swr = im, lsw
        cols.append(jnp.sum(un * imr, axis=1, keepdims=True) - swr)
    out[...] = jnp.concatenate(cols, axis=1)


_dense_call = pl.pallas_call(
    _tc_dense,
    out_shape=jax.ShapeDtypeStruct((_B, 4), jnp.float32),
)


def kernel(user_id, user_cat1, user_cat2, user_cat3, item_id, item_cat1,
           sw_idx, user_tables, item_tables, sw_table,
           u_W1, u_b1, u_g1, u_be1, u_W2, u_b2, u_g2, u_be2,
           i_W1, i_b1, i_g1, i_be1, i_W2, i_b2, i_g2, i_be2):
    utf = jnp.transpose(user_tables, (0, 2, 1)).reshape(-1)
    itf = jnp.transpose(item_tables, (0, 2, 1)).reshape(-1)
    swf = sw_table.reshape(-1)
    d_off = jnp.arange(_D, dtype=jnp.int32) * _V
    u_idx = jnp.stack([user_id, user_cat1, user_cat2, user_cat3])
    uwidx = (u_idx[:, :, None] + d_off
             + (jnp.arange(4, dtype=jnp.int32) * 16 * _V)[:, None, None]
             ).reshape(4, _B * _D)
    i_idx = jnp.stack([item_id, item_cat1])
    iwidx = (i_idx[:, :, None] + d_off
             + (jnp.arange(2, dtype=jnp.int32) * 16 * _V)[:, None, None]
             ).reshape(2, _B * _D)
    u4f, i2f, sw = _gather_call()(utf, itf, swf, uwidx, iwidx, sw_idx)
    out = _dense_call(
        u4f.reshape(4, _B, _D), i2f.reshape(2, _B, _D), sw.reshape(_B, 1),
        u_W1, u_b1.reshape(1, -1), u_g1.reshape(1, -1), u_be1.reshape(1, -1),
        u_W2, u_b2.reshape(1, -1), u_g2.reshape(1, -1), u_be2.reshape(1, -1),
        i_W1, i_b1.reshape(1, -1), i_g1.reshape(1, -1), i_be1.reshape(1, -1),
        i_W2, i_b2.reshape(1, -1), i_g2.reshape(1, -1), i_be2.reshape(1, -1))
    return out
